# baseline (device time: 95284 ns/iter reference)
import jax
import jax.numpy as jnp
from jax import lax
from jax.experimental import pallas as pl
from jax.experimental.pallas import tpu as pltpu

N_DEV = 8
N_SUB = 4
N_STREAMS = 2 * N_SUB


def kernel(x, w_mat, scale_x, scale_w):
    m_per, k = x.shape
    _, n_per = w_mat.shape
    m_glob = N_DEV * m_per
    m_q = m_per // N_STREAMS

    x = x.astype(jnp.float8_e5m2)
    w_mat = w_mat.astype(jnp.float8_e5m2)

    def body(x_ref, w_ref, sx_ref, sw_ref, out_ref, xg_ref, send_sems, recv_sems):
        my = lax.axis_index("i")
        left = lax.rem(my + N_DEV - 1, N_DEV)
        right = lax.rem(my + 1, N_DEV)

        barrier_sem = pltpu.get_barrier_semaphore()
        for nbr in (left, right):
            pl.semaphore_signal(
                barrier_sem, inc=1,
                device_id=(nbr,), device_id_type=pl.DeviceIdType.MESH,
            )
        pl.semaphore_wait(barrier_sem, 2)

        scale = sx_ref[0] * sw_ref[0]

        def store(row0, nrows, blk):
            acc = jnp.dot(blk, w_ref[...], preferred_element_type=jnp.float32)
            out_ref[pl.ds(row0, nrows), :] = jnp.maximum(acc * scale, 0.0)

        def compute(row0, nrows):
            store(row0, nrows, xg_ref[pl.ds(row0, nrows), :])

        def stream_cfg(si):
            if si < N_SUB:
                return right, -1
            return left, +1

        def make(si, h):
            dev, sign = stream_cfg(si)
            o = lax.rem(my + sign * h + N_DEV, N_DEV)
            sl = xg_ref.at[pl.ds(o * m_per + si * m_q, m_q), :]
            src = x_ref.at[pl.ds(si * m_q, m_q), :] if h == 0 else sl
            return pltpu.make_async_remote_copy(
                src_ref=src, dst_ref=sl,
                send_sem=send_sems.at[si, h], recv_sem=recv_sems.at[si, h],
                device_id=(dev,), device_id_type=pl.DeviceIdType.MESH,
            )

        descs = [[make(si, h) for h in range(N_DEV - 1)] for si in range(N_STREAMS)]

        for si in range(N_STREAMS):
            descs[si][0].start()
        store(my * m_per, m_per, x_ref[...])

        for h in range(N_DEV - 1):
            for j in range(N_SUB):
                pair = (j, N_SUB + j)
                for si in pair:
                    descs[si][h].wait_recv()
                    if h + 1 < N_DEV - 1:
                        descs[si][h + 1].start()
                for si in pair:
                    _, sign = stream_cfg(si)
                    o = lax.rem(my + sign * (h + 1) + N_DEV, N_DEV)
                    compute(o * m_per + si * m_q, m_q)

        for si in range(N_STREAMS):
            for h in range(N_DEV - 1):
                descs[si][h].wait_send()

    return pl.pallas_call(
        body,
        out_shape=jax.ShapeDtypeStruct((m_glob, n_per), jnp.float32),
        in_specs=[
            pl.BlockSpec(memory_space=pltpu.VMEM),
            pl.BlockSpec(memory_space=pltpu.VMEM),
            pl.BlockSpec(memory_space=pltpu.SMEM),
            pl.BlockSpec(memory_space=pltpu.SMEM),
        ],
        out_specs=pl.BlockSpec(memory_space=pltpu.VMEM),
        scratch_shapes=[
            pltpu.VMEM((m_glob, k), x.dtype),
            pltpu.SemaphoreType.DMA((N_STREAMS, N_DEV - 1)),
            pltpu.SemaphoreType.DMA((N_STREAMS, N_DEV - 1)),
        ],
        compiler_params=pltpu.CompilerParams(collective_id=0),
    )(x, w_mat, scale_x, scale_w)


# device time: 93929 ns/iter; 1.0144x vs baseline; 1.0144x over previous
import jax
import jax.numpy as jnp
from jax import lax
from jax.experimental import pallas as pl
from jax.experimental.pallas import tpu as pltpu

N_DEV = 8
N_SUB = 4
N_STREAMS = 2 * N_SUB


def kernel(x, w_mat, scale_x, scale_w):
    m_per, k = x.shape
    _, n_per = w_mat.shape
    m_glob = N_DEV * m_per
    m_q = m_per // N_STREAMS

    def body(x_ref, w_ref, sx_ref, sw_ref, out_ref, xg_ref, x8_ref, w8_ref,
             send_sems, recv_sems):
        my = lax.axis_index("i")
        left = lax.rem(my + N_DEV - 1, N_DEV)
        right = lax.rem(my + 1, N_DEV)

        barrier_sem = pltpu.get_barrier_semaphore()
        for nbr in (left, right):
            pl.semaphore_signal(
                barrier_sem, inc=1,
                device_id=(nbr,), device_id_type=pl.DeviceIdType.MESH,
            )
        pl.semaphore_wait(barrier_sem, 2)

        scale = sx_ref[0] * sw_ref[0]

        def store(row0, nrows, blk):
            acc = jnp.dot(blk, w8_ref[...], preferred_element_type=jnp.float32)
            out_ref[pl.ds(row0, nrows), :] = jnp.maximum(acc * scale, 0.0)

        def compute(row0, nrows):
            store(row0, nrows, xg_ref[pl.ds(row0, nrows), :])

        def stream_cfg(si):
            if si < N_SUB:
                return right, -1
            return left, +1

        def make(si, h):
            dev, sign = stream_cfg(si)
            o = lax.rem(my + sign * h + N_DEV, N_DEV)
            sl = xg_ref.at[pl.ds(o * m_per + si * m_q, m_q), :]
            src = x8_ref.at[pl.ds(si * m_q, m_q), :] if h == 0 else sl
            return pltpu.make_async_remote_copy(
                src_ref=src, dst_ref=sl,
                send_sem=send_sems.at[si, h], recv_sem=recv_sems.at[si, h],
                device_id=(dev,), device_id_type=pl.DeviceIdType.MESH,
            )

        descs = [[make(si, h) for h in range(N_DEV - 1)] for si in range(N_STREAMS)]

        for si in range(N_STREAMS):
            x8_ref[pl.ds(si * m_q, m_q), :] = (
                x_ref[pl.ds(si * m_q, m_q), :].astype(jnp.float8_e5m2))
            descs[si][0].start()
        w8_ref[...] = w_ref[...].astype(jnp.float8_e5m2)
        store(my * m_per, m_per, x8_ref[...])

        for h in range(N_DEV - 1):
            for j in range(N_SUB):
                pair = (j, N_SUB + j)
                for si in pair:
                    descs[si][h].wait_recv()
                    if h + 1 < N_DEV - 1:
                        descs[si][h + 1].start()
                for si in pair:
                    _, sign = stream_cfg(si)
                    o = lax.rem(my + sign * (h + 1) + N_DEV, N_DEV)
                    compute(o * m_per + si * m_q, m_q)

        for si in range(N_STREAMS):
            for h in range(N_DEV - 1):
                descs[si][h].wait_send()

    return pl.pallas_call(
        body,
        out_shape=jax.ShapeDtypeStruct((m_glob, n_per), jnp.float32),
        in_specs=[
            pl.BlockSpec(memory_space=pltpu.VMEM),
            pl.BlockSpec(memory_space=pltpu.VMEM),
            pl.BlockSpec(memory_space=pltpu.SMEM),
            pl.BlockSpec(memory_space=pltpu.SMEM),
        ],
        out_specs=pl.BlockSpec(memory_space=pltpu.VMEM),
        scratch_shapes=[
            pltpu.VMEM((m_glob, k), jnp.float8_e5m2),
            pltpu.VMEM((m_per, k), jnp.float8_e5m2),
            pltpu.VMEM((k, n_per), jnp.float8_e5m2),
            pltpu.SemaphoreType.DMA((N_STREAMS, N_DEV - 1)),
            pltpu.SemaphoreType.DMA((N_STREAMS, N_DEV - 1)),
        ],
        compiler_params=pltpu.CompilerParams(collective_id=0),
    )(x, w_mat, scale_x, scale_w)
